# baseline (speedup 1.0000x reference)
import functools
import math

import jax
import jax.numpy as jnp
from jax import lax
from jax.experimental import pallas as pl
from jax.experimental.pallas import tpu as pltpu
from jax.experimental.pallas import tpu_sc as plsc

_NODE_NUM = 100000
_NUM_LAYER = 2
_L1 = _NUM_LAYER + 1
_RP_DIM = int(10 * math.log(3200000))
_B = 4096
_NW = 32
_ROWS = 2 * _L1 * _B
_RPW = _ROWS // _NW
_CHUNK = 128
_NCH = _RPW // _CHUNK

_BS = 256
_GRID = _B // _BS


def _sc_gather(table, idx):
    mesh = plsc.VectorSubcoreMesh(core_axis_name="c", subcore_axis_name="s")

    @functools.partial(
        pl.kernel,
        out_type=jax.ShapeDtypeStruct((_ROWS, _RP_DIM), jnp.float32),
        mesh=mesh,
        scratch_types=[
            pltpu.VMEM((_CHUNK,), jnp.int32),
            pltpu.VMEM((_CHUNK, _RP_DIM), jnp.float32),
            pltpu.SemaphoreType.DMA,
        ],
        compiler_params=pltpu.CompilerParams(use_tc_tiling_on_sc=True),
    )
    def gather_kernel(table_hbm, idx_hbm, out_hbm, idx_v, rows_v, sem):
        wid = lax.axis_index("s") * 2 + lax.axis_index("c")
        base = wid * _RPW

        def chunk(j, carry):
            off = base + j * _CHUNK
            pltpu.sync_copy(idx_hbm.at[pl.ds(off, _CHUNK)], idx_v)
            for g in range(_CHUNK // 16):
                vec = idx_v[pl.ds(g * 16, 16)]
                for i in range(16):
                    pltpu.async_copy(
                        table_hbm.at[pl.ds(vec[i], 1)],
                        rows_v.at[pl.ds(g * 16 + i, 1)], sem)
            pltpu.make_async_copy(
                table_hbm.at[pl.ds(0, _CHUNK)], rows_v, sem).wait()
            pltpu.sync_copy(rows_v, out_hbm.at[pl.ds(off, _CHUNK)])
            return carry

        lax.fori_loop(0, _NCH, chunk, 0)

    return gather_kernel(table, idx)


def _tc_body(rows_ref, w1_ref, b1_ref, w2_ref, b2_ref, out_ref):
    r = rows_ref[...]
    s = [r[k] for k in range(_L1)]
    d = [r[_L1 + k] for k in range(_L1)]
    sd = [[jnp.sum(s[l] * d[m], axis=1) for m in range(_L1)] for l in range(_L1)]
    ss = [[jnp.sum(s[l] * s[m], axis=1) for m in range(_L1)] for l in range(_L1)]
    dd = [[jnp.sum(d[l] * d[m], axis=1) for m in range(_L1)] for l in range(_L1)]
    cols = (
        [sd[l][m] for l in range(_L1) for m in range(_L1)]
        + [sd[m][l] for l in range(_L1) for m in range(_L1)]
        + [ss[l][m] for l in range(_L1) for m in range(_L1)]
        + [dd[l][m] for l in range(_L1) for m in range(_L1)]
    )
    feat = jnp.stack(cols, axis=1)
    feat = jnp.log(jnp.maximum(feat, 0.0) + 1.0)
    h = jnp.dot(feat, w1_ref[...], preferred_element_type=jnp.float32)
    h = jnp.maximum(h + b1_ref[...], 0.0)
    out_ref[...] = jnp.sum(h * w2_ref[...], axis=1) + b2_ref[0, 0]


def _tc_compute(rows, w1, b1, w2t, b2):
    feat_dim = 4 * _L1 * _L1
    hidden = w1.shape[1]
    return pl.pallas_call(
        _tc_body,
        grid=(_GRID,),
        in_specs=[
            pl.BlockSpec((2 * _L1, _BS, _RP_DIM), lambda i: (0, i, 0)),
            pl.BlockSpec((feat_dim, hidden), lambda i: (0, 0)),
            pl.BlockSpec((1, hidden), lambda i: (0, 0)),
            pl.BlockSpec((1, hidden), lambda i: (0, 0)),
            pl.BlockSpec((1, 1), lambda i: (0, 0)),
        ],
        out_specs=pl.BlockSpec((_BS,), lambda i: (i,)),
        out_shape=jax.ShapeDtypeStruct((_B,), jnp.float32),
    )(rows, w1, b1, w2t, b2)


def kernel(P, W1, b1, W2, b2, src_node_ids, dst_node_ids):
    table = P.reshape(_L1 * _NODE_NUM, _RP_DIM)
    lofs = (jnp.arange(_L1, dtype=jnp.int32) * _NODE_NUM)[:, None]
    idx = jnp.concatenate(
        [src_node_ids[None, :].astype(jnp.int32) + lofs,
         dst_node_ids[None, :].astype(jnp.int32) + lofs], axis=0
    ).reshape(_ROWS)
    rows = _sc_gather(table, idx).reshape(2 * _L1, _B, _RP_DIM)
    return _tc_compute(rows, W1, b1.reshape(1, -1), W2.reshape(1, -1),
                       b2.reshape(1, 1))


# speedup vs baseline: 4.5043x; 4.5043x over previous
import functools
import math

import jax
import jax.numpy as jnp
from jax import lax
from jax.experimental import pallas as pl
from jax.experimental.pallas import tpu as pltpu
from jax.experimental.pallas import tpu_sc as plsc

_NODE_NUM = 100000
_NUM_LAYER = 2
_L1 = _NUM_LAYER + 1
_RP_DIM = int(10 * math.log(3200000))
_B = 4096
_NW = 32
_ROWS = 2 * _L1 * _B
_RPW = _ROWS // _NW
_CHUNK = 128
_NCH = _RPW // _CHUNK

_BS = 256
_GRID = _B // _BS


def _sc_gather(table, idx):
    mesh = plsc.VectorSubcoreMesh(core_axis_name="c", subcore_axis_name="s")

    @functools.partial(
        pl.kernel,
        out_type=jax.ShapeDtypeStruct((_ROWS, _RP_DIM), jnp.float32),
        mesh=mesh,
        scratch_types=[
            pltpu.VMEM((_CHUNK,), jnp.int32),
            pltpu.VMEM((_CHUNK, _RP_DIM), jnp.float32),
            pltpu.SemaphoreType.DMA,
        ],
        compiler_params=pltpu.CompilerParams(use_tc_tiling_on_sc=True),
    )
    def gather_kernel(table_hbm, idx_hbm, out_hbm, idx_v, rows_v, sem):
        wid = lax.axis_index("s") * 2 + lax.axis_index("c")
        for p in range(2 * _L1):
            off = p * _B + wid * _CHUNK
            pltpu.sync_copy(idx_hbm.at[pl.ds(off, _CHUNK)], idx_v)
            for g in range(_CHUNK // 16):
                vec = idx_v[pl.ds(g * 16, 16)]
                for i in range(16):
                    pltpu.async_copy(
                        table_hbm.at[p % _L1, pl.ds(vec[i], 1)],
                        rows_v.at[pl.ds(g * 16 + i, 1)], sem)
            pltpu.make_async_copy(
                table_hbm.at[0, pl.ds(0, _CHUNK)], rows_v, sem).wait()
            pltpu.sync_copy(rows_v, out_hbm.at[pl.ds(off, _CHUNK)])

    return gather_kernel(table, idx)


def _tc_body(rows_ref, w1_ref, b1_ref, w2_ref, b2_ref, out_ref):
    r = rows_ref[...]
    s = [r[k] for k in range(_L1)]
    d = [r[_L1 + k] for k in range(_L1)]
    sd = [[jnp.sum(s[l] * d[m], axis=1) for m in range(_L1)] for l in range(_L1)]
    ss = [[jnp.sum(s[l] * s[m], axis=1) for m in range(_L1)] for l in range(_L1)]
    dd = [[jnp.sum(d[l] * d[m], axis=1) for m in range(_L1)] for l in range(_L1)]
    cols = (
        [sd[l][m] for l in range(_L1) for m in range(_L1)]
        + [sd[m][l] for l in range(_L1) for m in range(_L1)]
        + [ss[l][m] for l in range(_L1) for m in range(_L1)]
        + [dd[l][m] for l in range(_L1) for m in range(_L1)]
    )
    feat = jnp.stack(cols, axis=1)
    feat = jnp.log(jnp.maximum(feat, 0.0) + 1.0)
    h = jnp.dot(feat, w1_ref[...], preferred_element_type=jnp.float32)
    h = jnp.maximum(h + b1_ref[...], 0.0)
    out_ref[...] = jnp.sum(h * w2_ref[...], axis=1) + b2_ref[0, 0]


def _tc_compute(rows, w1, b1, w2t, b2):
    feat_dim = 4 * _L1 * _L1
    hidden = w1.shape[1]
    return pl.pallas_call(
        _tc_body,
        grid=(_GRID,),
        in_specs=[
            pl.BlockSpec((2 * _L1, _BS, _RP_DIM), lambda i: (0, i, 0)),
            pl.BlockSpec((feat_dim, hidden), lambda i: (0, 0)),
            pl.BlockSpec((1, hidden), lambda i: (0, 0)),
            pl.BlockSpec((1, hidden), lambda i: (0, 0)),
            pl.BlockSpec((1, 1), lambda i: (0, 0)),
        ],
        out_specs=pl.BlockSpec((_BS,), lambda i: (i,)),
        out_shape=jax.ShapeDtypeStruct((_B,), jnp.float32),
    )(rows, w1, b1, w2t, b2)


def kernel(P, W1, b1, W2, b2, src_node_ids, dst_node_ids):
    idx = jnp.concatenate(
        [jnp.broadcast_to(src_node_ids.astype(jnp.int32), (_L1, _B)),
         jnp.broadcast_to(dst_node_ids.astype(jnp.int32), (_L1, _B))], axis=0
    ).reshape(_ROWS)
    rows = _sc_gather(P, idx).reshape(2 * _L1, _B, _RP_DIM)
    return _tc_compute(rows, W1, b1.reshape(1, -1), W2.reshape(1, -1),
                       b2.reshape(1, 1))


# speedup vs baseline: 7.1706x; 1.5920x over previous
import functools
import math

import jax
import jax.numpy as jnp
from jax import lax
from jax.experimental import pallas as pl
from jax.experimental.pallas import tpu as pltpu
from jax.experimental.pallas import tpu_sc as plsc

_NODE_NUM = 100000
_NUM_LAYER = 2
_L1 = _NUM_LAYER + 1
_RP_DIM = int(10 * math.log(3200000))
_DPAD = 152
_B = 4096
_IDS = 2 * _B
_NW = 32
_KMAX = 5

_BS = 256
_GRID = _B // _BS


def _sc_gather(table, ids):
    mesh = plsc.VectorSubcoreMesh(core_axis_name="c", subcore_axis_name="s")

    @functools.partial(
        pl.kernel,
        out_type=jax.ShapeDtypeStruct((_L1 * _DPAD, _IDS), jnp.float32),
        mesh=mesh,
        scratch_types=[
            pltpu.VMEM((_IDS,), jnp.int32),
            pltpu.VMEM((_NODE_NUM,), jnp.float32),
            pltpu.VMEM((_IDS,), jnp.float32),
            pltpu.SemaphoreType.DMA,
        ],
        compiler_params=pltpu.CompilerParams(
            use_tc_tiling_on_sc=True, needs_layout_passes=False),
    )
    def gather_kernel(table_hbm, ids_hbm, out_hbm, ids_v, row_v, out_v, sem):
        wid = lax.axis_index("s") * 2 + lax.axis_index("c")
        pltpu.sync_copy(ids_hbm, ids_v)
        for l in range(_L1):
            for k in range(_KMAX):
                d = wid + k * _NW

                @pl.when(d < _RP_DIM)
                def _():
                    pltpu.sync_copy(table_hbm.at[l, d], row_v)

                    def gather16(g, carry):
                        vec = ids_v[pl.ds(g * 16, 16)]
                        out_v[pl.ds(g * 16, 16)] = plsc.load_gather(
                            row_v, [vec])
                        return carry

                    lax.fori_loop(0, _IDS // 16, gather16, 0, unroll=4)
                    pltpu.sync_copy(out_v, out_hbm.at[l * _DPAD + d])

    return gather_kernel(table, ids)


def _tc_body(rs_ref, rd_ref, w1t_ref, b1_ref, w2_ref, b2_ref, out_ref):
    rs = rs_ref[...]
    rd = rd_ref[...]
    s = [rs[l * _DPAD:l * _DPAD + _RP_DIM] for l in range(_L1)]
    d = [rd[l * _DPAD:l * _DPAD + _RP_DIM] for l in range(_L1)]
    sd = [[jnp.sum(s[l] * d[m], axis=0) for m in range(_L1)] for l in range(_L1)]
    ss = [[jnp.sum(s[l] * s[m], axis=0) for m in range(_L1)] for l in range(_L1)]
    dd = [[jnp.sum(d[l] * d[m], axis=0) for m in range(_L1)] for l in range(_L1)]
    rows = (
        [sd[l][m] for l in range(_L1) for m in range(_L1)]
        + [sd[m][l] for l in range(_L1) for m in range(_L1)]
        + [ss[l][m] for l in range(_L1) for m in range(_L1)]
        + [dd[l][m] for l in range(_L1) for m in range(_L1)]
    )
    feat = jnp.stack(rows, axis=0)
    feat = jnp.log(jnp.maximum(feat, 0.0) + 1.0)
    h = jnp.dot(w1t_ref[...], feat, preferred_element_type=jnp.float32)
    h = jnp.maximum(h + b1_ref[...], 0.0)
    out_ref[...] = jnp.sum(h * w2_ref[...], axis=0) + b2_ref[0, 0]


def _tc_compute(g, w1t, b1, w2, b2):
    hidden = w1t.shape[0]
    feat_dim = 4 * _L1 * _L1
    nsrc = _B // _BS
    return pl.pallas_call(
        _tc_body,
        grid=(_GRID,),
        in_specs=[
            pl.BlockSpec((_L1 * _DPAD, _BS), lambda i: (0, i)),
            pl.BlockSpec((_L1 * _DPAD, _BS), lambda i: (0, nsrc + i)),
            pl.BlockSpec((hidden, feat_dim), lambda i: (0, 0)),
            pl.BlockSpec((hidden, 1), lambda i: (0, 0)),
            pl.BlockSpec((hidden, 1), lambda i: (0, 0)),
            pl.BlockSpec((1, 1), lambda i: (0, 0)),
        ],
        out_specs=pl.BlockSpec((_BS,), lambda i: (i,)),
        out_shape=jax.ShapeDtypeStruct((_B,), jnp.float32),
    )(g, g, w1t, b1, w2, b2)


def kernel(P, W1, b1, W2, b2, src_node_ids, dst_node_ids):
    table = jnp.transpose(P, (0, 2, 1))
    ids = jnp.concatenate([src_node_ids.astype(jnp.int32),
                           dst_node_ids.astype(jnp.int32)])
    g = _sc_gather(table, ids)
    return _tc_compute(g, W1.T, b1.reshape(-1, 1), W2.reshape(-1, 1),
                       b2.reshape(1, 1))


# speedup vs baseline: 7.5767x; 1.0566x over previous
import functools
import math

import jax
import jax.numpy as jnp
from jax import lax
from jax.experimental import pallas as pl
from jax.experimental.pallas import tpu as pltpu
from jax.experimental.pallas import tpu_sc as plsc

_NODE_NUM = 100000
_NUM_LAYER = 2
_L1 = _NUM_LAYER + 1
_RP_DIM = int(10 * math.log(3200000))
_DPAD = 152
_B = 4096
_IDS = 2 * _B
_NW = 32
_KMAX = 5

_BS = 256
_GRID = _B // _BS


def _sc_gather(table, ids):
    mesh = plsc.VectorSubcoreMesh(core_axis_name="c", subcore_axis_name="s")

    @functools.partial(
        pl.kernel,
        out_type=jax.ShapeDtypeStruct((_L1 * _DPAD, _IDS), jnp.float32),
        mesh=mesh,
        scratch_types=[
            pltpu.VMEM((_IDS,), jnp.int32),
            pltpu.VMEM((_NODE_NUM,), jnp.float32),
            pltpu.VMEM((_IDS,), jnp.float32),
            pltpu.SemaphoreType.DMA,
        ],
        compiler_params=pltpu.CompilerParams(
            use_tc_tiling_on_sc=True, needs_layout_passes=False),
    )
    def gather_kernel(table_hbm, ids_hbm, out_hbm, ids_v, row_v, out_v, sem):
        wid = lax.axis_index("s") * 2 + lax.axis_index("c")
        pltpu.sync_copy(ids_hbm, ids_v)
        for l in range(_L1):
            for k in range(_KMAX):
                d = (wid + 11 * l) % _NW + k * _NW

                @pl.when(d < _RP_DIM)
                def _():
                    pltpu.sync_copy(table_hbm.at[l, d], row_v)

                    def gather16(g, carry):
                        vec = ids_v[pl.ds(g * 16, 16)]
                        out_v[pl.ds(g * 16, 16)] = plsc.load_gather(
                            row_v, [vec])
                        return carry

                    lax.fori_loop(0, _IDS // 16, gather16, 0, unroll=4)
                    pltpu.sync_copy(out_v, out_hbm.at[l * _DPAD + d])

    return gather_kernel(table, ids)


def _tc_body(rs_ref, rd_ref, w1t_ref, b1_ref, w2_ref, b2_ref, out_ref):
    rs = rs_ref[...]
    rd = rd_ref[...]
    s = [rs[l * _DPAD:l * _DPAD + _RP_DIM] for l in range(_L1)]
    d = [rd[l * _DPAD:l * _DPAD + _RP_DIM] for l in range(_L1)]
    sd = [[jnp.sum(s[l] * d[m], axis=0) for m in range(_L1)] for l in range(_L1)]
    ss = [[jnp.sum(s[l] * s[m], axis=0) for m in range(_L1)] for l in range(_L1)]
    dd = [[jnp.sum(d[l] * d[m], axis=0) for m in range(_L1)] for l in range(_L1)]
    rows = (
        [sd[l][m] for l in range(_L1) for m in range(_L1)]
        + [sd[m][l] for l in range(_L1) for m in range(_L1)]
        + [ss[l][m] for l in range(_L1) for m in range(_L1)]
        + [dd[l][m] for l in range(_L1) for m in range(_L1)]
    )
    feat = jnp.stack(rows, axis=0)
    feat = jnp.log(jnp.maximum(feat, 0.0) + 1.0)
    h = jnp.dot(w1t_ref[...], feat, preferred_element_type=jnp.float32)
    h = jnp.maximum(h + b1_ref[...], 0.0)
    out_ref[...] = jnp.sum(h * w2_ref[...], axis=0) + b2_ref[0, 0]


def _tc_compute(g, w1t, b1, w2, b2):
    hidden = w1t.shape[0]
    feat_dim = 4 * _L1 * _L1
    nsrc = _B // _BS
    return pl.pallas_call(
        _tc_body,
        grid=(_GRID,),
        in_specs=[
            pl.BlockSpec((_L1 * _DPAD, _BS), lambda i: (0, i)),
            pl.BlockSpec((_L1 * _DPAD, _BS), lambda i: (0, nsrc + i)),
            pl.BlockSpec((hidden, feat_dim), lambda i: (0, 0)),
            pl.BlockSpec((hidden, 1), lambda i: (0, 0)),
            pl.BlockSpec((hidden, 1), lambda i: (0, 0)),
            pl.BlockSpec((1, 1), lambda i: (0, 0)),
        ],
        out_specs=pl.BlockSpec((_BS,), lambda i: (i,)),
        out_shape=jax.ShapeDtypeStruct((_B,), jnp.float32),
    )(g, g, w1t, b1, w2, b2)


def kernel(P, W1, b1, W2, b2, src_node_ids, dst_node_ids):
    table = jnp.transpose(P, (0, 2, 1))
    ids = jnp.concatenate([src_node_ids.astype(jnp.int32),
                           dst_node_ids.astype(jnp.int32)])
    g = _sc_gather(table, ids)
    return _tc_compute(g, W1.T, b1.reshape(-1, 1), W2.reshape(-1, 1),
                       b2.reshape(1, 1))


# speedup vs baseline: 7.7590x; 1.0241x over previous
import functools
import math

import jax
import jax.numpy as jnp
from jax import lax
from jax.experimental import pallas as pl
from jax.experimental.pallas import tpu as pltpu
from jax.experimental.pallas import tpu_sc as plsc

_NODE_NUM = 100000
_NUM_LAYER = 2
_L1 = _NUM_LAYER + 1
_RP_DIM = int(10 * math.log(3200000))
_DPAD = 152
_B = 4096
_IDS = 2 * _B
_NW = 32
_KMAX = 5

_BS = 256
_GRID = _B // _BS


def _sc_gather(table, ids):
    mesh = plsc.VectorSubcoreMesh(core_axis_name="c", subcore_axis_name="s")

    @functools.partial(
        pl.kernel,
        out_type=jax.ShapeDtypeStruct((_L1 * _DPAD, _IDS), jnp.float32),
        mesh=mesh,
        scratch_types=[
            pltpu.VMEM((_IDS,), jnp.int32),
            pltpu.VMEM((_NODE_NUM,), jnp.float32),
            pltpu.VMEM((_IDS,), jnp.float32),
            pltpu.VMEM((_IDS,), jnp.float32),
            pltpu.SemaphoreType.DMA,
            pltpu.SemaphoreType.DMA,
            pltpu.SemaphoreType.DMA,
        ],
        compiler_params=pltpu.CompilerParams(
            use_tc_tiling_on_sc=True, needs_layout_passes=False),
    )
    def gather_kernel(table_hbm, ids_hbm, out_hbm, ids_v, row_v, out_a,
                      out_b, sem, sem_oa, sem_ob):
        wid = lax.axis_index("s") * 2 + lax.axis_index("c")
        obufs = (out_a, out_b)
        osems = (sem_oa, sem_ob)
        pltpu.sync_copy(ids_hbm, ids_v)
        prev = [None, None]

        def drain(par):
            pl_, pk_ = prev[par]
            pd = (wid + 11 * pl_) % _NW + pk_ * _NW

            def w():
                pltpu.make_async_copy(
                    obufs[par], out_hbm.at[pl_ * _DPAD + pd],
                    osems[par]).wait()

            if pk_ < _KMAX - 1:
                w()
            else:
                pl.when(pd < _RP_DIM)(w)

        t = 0
        for l in range(_L1):
            for k in range(_KMAX):
                d = (wid + 11 * l) % _NW + k * _NW
                par = t % 2
                ob, osem = obufs[par], osems[par]
                if prev[par] is not None:
                    drain(par)

                def row_work(l=l, d=d, ob=ob, osem=osem):
                    pltpu.sync_copy(table_hbm.at[l, d], row_v)

                    def gather16(g, carry):
                        vec = ids_v[pl.ds(g * 16, 16)]
                        ob[pl.ds(g * 16, 16)] = plsc.load_gather(
                            row_v, [vec])
                        return carry

                    lax.fori_loop(0, _IDS // 16, gather16, 0, unroll=8)
                    pltpu.async_copy(ob, out_hbm.at[l * _DPAD + d], osem)

                if k < _KMAX - 1:
                    row_work()
                else:
                    pl.when(d < _RP_DIM)(row_work)
                prev[par] = (l, k)
                t += 1
        drain(0)
        drain(1)

    return gather_kernel(table, ids)


def _tc_body(rs_ref, rd_ref, w1t_ref, b1_ref, w2_ref, b2_ref, out_ref):
    rs = rs_ref[...]
    rd = rd_ref[...]
    s = [rs[l * _DPAD:l * _DPAD + _RP_DIM] for l in range(_L1)]
    d = [rd[l * _DPAD:l * _DPAD + _RP_DIM] for l in range(_L1)]
    sd = [[jnp.sum(s[l] * d[m], axis=0) for m in range(_L1)] for l in range(_L1)]
    ss = [[jnp.sum(s[l] * s[m], axis=0) for m in range(_L1)] for l in range(_L1)]
    dd = [[jnp.sum(d[l] * d[m], axis=0) for m in range(_L1)] for l in range(_L1)]
    rows = (
        [sd[l][m] for l in range(_L1) for m in range(_L1)]
        + [sd[m][l] for l in range(_L1) for m in range(_L1)]
        + [ss[l][m] for l in range(_L1) for m in range(_L1)]
        + [dd[l][m] for l in range(_L1) for m in range(_L1)]
    )
    feat = jnp.stack(rows, axis=0)
    feat = jnp.log(jnp.maximum(feat, 0.0) + 1.0)
    h = jnp.dot(w1t_ref[...], feat, preferred_element_type=jnp.float32)
    h = jnp.maximum(h + b1_ref[...], 0.0)
    out_ref[...] = jnp.sum(h * w2_ref[...], axis=0) + b2_ref[0, 0]


def _tc_compute(g, w1t, b1, w2, b2):
    hidden = w1t.shape[0]
    feat_dim = 4 * _L1 * _L1
    nsrc = _B // _BS
    return pl.pallas_call(
        _tc_body,
        grid=(_GRID,),
        in_specs=[
            pl.BlockSpec((_L1 * _DPAD, _BS), lambda i: (0, i)),
            pl.BlockSpec((_L1 * _DPAD, _BS), lambda i: (0, nsrc + i)),
            pl.BlockSpec((hidden, feat_dim), lambda i: (0, 0)),
            pl.BlockSpec((hidden, 1), lambda i: (0, 0)),
            pl.BlockSpec((hidden, 1), lambda i: (0, 0)),
            pl.BlockSpec((1, 1), lambda i: (0, 0)),
        ],
        out_specs=pl.BlockSpec((_BS,), lambda i: (i,)),
        out_shape=jax.ShapeDtypeStruct((_B,), jnp.float32),
    )(g, g, w1t, b1, w2, b2)


def kernel(P, W1, b1, W2, b2, src_node_ids, dst_node_ids):
    table = jnp.transpose(P, (0, 2, 1))
    ids = jnp.concatenate([src_node_ids.astype(jnp.int32),
                           dst_node_ids.astype(jnp.int32)])
    g = _sc_gather(table, ids)
    return _tc_compute(g, W1.T, b1.reshape(-1, 1), W2.reshape(-1, 1),
                       b2.reshape(1, 1))


# speedup vs baseline: 7.9609x; 1.0260x over previous
import functools
import math

import jax
import jax.numpy as jnp
from jax import lax
from jax.experimental import pallas as pl
from jax.experimental.pallas import tpu as pltpu
from jax.experimental.pallas import tpu_sc as plsc

_NODE_NUM = 100000
_NUM_LAYER = 2
_L1 = _NUM_LAYER + 1
_RP_DIM = int(10 * math.log(3200000))
_DPAD = 152
_B = 4096
_IDS = 2 * _B
_NW = 32
_KMAX = 5

_BS = 512
_GRID = _B // _BS


def _sc_gather(table, ids):
    mesh = plsc.VectorSubcoreMesh(core_axis_name="c", subcore_axis_name="s")

    @functools.partial(
        pl.kernel,
        out_type=jax.ShapeDtypeStruct((_L1 * _DPAD, _IDS), jnp.float32),
        mesh=mesh,
        scratch_types=[
            pltpu.VMEM((_IDS,), jnp.int32),
            pltpu.VMEM((_NODE_NUM,), jnp.float32),
            pltpu.VMEM((_IDS,), jnp.float32),
            pltpu.VMEM((_IDS,), jnp.float32),
            pltpu.SemaphoreType.DMA,
            pltpu.SemaphoreType.DMA,
            pltpu.SemaphoreType.DMA,
        ],
        compiler_params=pltpu.CompilerParams(
            use_tc_tiling_on_sc=True, needs_layout_passes=False),
    )
    def gather_kernel(table_hbm, ids_hbm, out_hbm, ids_v, row_v, out_a,
                      out_b, sem, sem_oa, sem_ob):
        wid = lax.axis_index("s") * 2 + lax.axis_index("c")
        obufs = (out_a, out_b)
        osems = (sem_oa, sem_ob)
        pltpu.sync_copy(ids_hbm, ids_v)
        prev = [None, None]

        def drain(par):
            pl_, pk_ = prev[par]
            pd = (wid + 11 * pl_) % _NW + pk_ * _NW

            def w():
                pltpu.make_async_copy(
                    obufs[par], out_hbm.at[pl_ * _DPAD + pd],
                    osems[par]).wait()

            if pk_ < _KMAX - 1:
                w()
            else:
                pl.when(pd < _RP_DIM)(w)

        t = 0
        for l in range(_L1):
            for k in range(_KMAX):
                d = (wid + 11 * l) % _NW + k * _NW
                par = t % 2
                ob, osem = obufs[par], osems[par]
                if prev[par] is not None:
                    drain(par)

                def row_work(l=l, d=d, ob=ob, osem=osem):
                    pltpu.sync_copy(table_hbm.at[l, d], row_v)

                    def gather16(g, carry):
                        vec = ids_v[pl.ds(g * 16, 16)]
                        ob[pl.ds(g * 16, 16)] = plsc.load_gather(
                            row_v, [vec])
                        return carry

                    lax.fori_loop(0, _IDS // 16, gather16, 0, unroll=8)
                    pltpu.async_copy(ob, out_hbm.at[l * _DPAD + d], osem)

                if k < _KMAX - 1:
                    row_work()
                else:
                    pl.when(d < _RP_DIM)(row_work)
                prev[par] = (l, k)
                t += 1
        drain(0)
        drain(1)

    return gather_kernel(table, ids)


def _tc_body(rs_ref, rd_ref, w1t_ref, b1_ref, w2_ref, b2_ref, out_ref):
    rs = rs_ref[...]
    rd = rd_ref[...]
    s = [rs[l * _DPAD:l * _DPAD + _RP_DIM] for l in range(_L1)]
    d = [rd[l * _DPAD:l * _DPAD + _RP_DIM] for l in range(_L1)]
    sd = [[jnp.sum(s[l] * d[m], axis=0) for m in range(_L1)] for l in range(_L1)]
    ss = [[jnp.sum(s[l] * s[m], axis=0) for m in range(_L1)] for l in range(_L1)]
    dd = [[jnp.sum(d[l] * d[m], axis=0) for m in range(_L1)] for l in range(_L1)]
    rows = (
        [sd[l][m] for l in range(_L1) for m in range(_L1)]
        + [sd[m][l] for l in range(_L1) for m in range(_L1)]
        + [ss[l][m] for l in range(_L1) for m in range(_L1)]
        + [dd[l][m] for l in range(_L1) for m in range(_L1)]
    )
    feat = jnp.stack(rows, axis=0)
    feat = jnp.log(jnp.maximum(feat, 0.0) + 1.0)
    h = jnp.dot(w1t_ref[...], feat, preferred_element_type=jnp.float32)
    h = jnp.maximum(h + b1_ref[...], 0.0)
    out_ref[...] = jnp.sum(h * w2_ref[...], axis=0) + b2_ref[0, 0]


def _tc_compute(g, w1t, b1, w2, b2):
    hidden = w1t.shape[0]
    feat_dim = 4 * _L1 * _L1
    nsrc = _B // _BS
    return pl.pallas_call(
        _tc_body,
        grid=(_GRID,),
        in_specs=[
            pl.BlockSpec((_L1 * _DPAD, _BS), lambda i: (0, i)),
            pl.BlockSpec((_L1 * _DPAD, _BS), lambda i: (0, nsrc + i)),
            pl.BlockSpec((hidden, feat_dim), lambda i: (0, 0)),
            pl.BlockSpec((hidden, 1), lambda i: (0, 0)),
            pl.BlockSpec((hidden, 1), lambda i: (0, 0)),
            pl.BlockSpec((1, 1), lambda i: (0, 0)),
        ],
        out_specs=pl.BlockSpec((_BS,), lambda i: (i,)),
        out_shape=jax.ShapeDtypeStruct((_B,), jnp.float32),
    )(g, g, w1t, b1, w2, b2)


def kernel(P, W1, b1, W2, b2, src_node_ids, dst_node_ids):
    table = jnp.transpose(P, (0, 2, 1))
    ids = jnp.concatenate([src_node_ids.astype(jnp.int32),
                           dst_node_ids.astype(jnp.int32)])
    g = _sc_gather(table, ids)
    return _tc_compute(g, W1.T, b1.reshape(-1, 1), W2.reshape(-1, 1),
                       b2.reshape(1, 1))
